# baseline (speedup 1.0000x reference)
import functools

import jax
import jax.numpy as jnp
from jax import lax
from jax.experimental import pallas as pl
from jax.experimental.pallas import tpu as pltpu
from jax.experimental.pallas import tpu_sc as plsc

IN_LENGTH = 16
ENC = 256
EMB = 64
CLASS_NUM = 3
B = 256
G = 32
N = B * G

SB = 64
AB = SB * G
SG = 8


_SC_NC = 2
_SC_NS = 16
_SC_NW = _SC_NC * _SC_NS


def _sc_gather(table, idx):
    rows, d = table.shape
    b_per_w = rows // _SC_NW
    mesh = plsc.VectorSubcoreMesh(core_axis_name="c", subcore_axis_name="s")

    @functools.partial(
        pl.kernel, mesh=mesh,
        out_type=jax.ShapeDtypeStruct((rows, d), table.dtype),
        scratch_types=[
            pltpu.VMEM((b_per_w,), jnp.int32),
            pltpu.VMEM((b_per_w, d), table.dtype),
            pltpu.SemaphoreType.DMA,
        ],
    )
    def k(table_hbm, idx_hbm, out_hbm, idx_v, rows_v, sem):
        wid = lax.axis_index("s") * _SC_NC + lax.axis_index("c")
        base = wid * b_per_w
        pltpu.sync_copy(idx_hbm.at[pl.ds(base, b_per_w)], idx_v)
        pltpu.async_copy(table_hbm.at[idx_v], rows_v, sem).wait()
        pltpu.sync_copy(rows_v, out_hbm.at[pl.ds(base, b_per_w)])

    return k(table, idx)


def _lrelu(v):
    return jnp.maximum(v, 0.1 * v)


def _sigmoid(v):
    return 0.5 * jnp.tanh(0.5 * v) + 0.5


def _fused_kernel(F_ref, Fg_ref, Wbig_ref, bemb_ref,
                  WihTf_ref, WhhTf_ref, bihf_ref, bhhf_ref,
                  WihTb_ref, WhhTb_ref, bihb_ref, bhhb_ref,
                  WmlpT_ref, bmlp_ref, Wsp1T_ref, bsp1_ref,
                  Wsp2T_ref, bsp2_ref, Wop1T_ref, bop1_ref,
                  Wop2T_ref, bop2_ref,
                  logits_ref, full_ref):
    f32 = jnp.float32
    bf = jnp.bfloat16
    F = F_ref[...]
    emb = _lrelu(jnp.dot(F.astype(bf), Wbig_ref[...].astype(bf),
                         preferred_element_type=f32)
                 + bemb_ref[...])
    emb = emb.astype(bf)

    def gru_step(h, t, WihT_ref, WhhT_ref, bih_ref, bhh_ref):
        e_t = emb[:, t * 128:(t + 1) * 128]
        gi = jnp.dot(e_t, WihT_ref[...].astype(bf),
                     preferred_element_type=f32) + bih_ref[...]
        gh = jnp.dot(h.astype(bf), WhhT_ref[...].astype(bf),
                     preferred_element_type=f32) + bhh_ref[...]
        r = _sigmoid(gi[:, :ENC] + gh[:, :ENC])
        z = _sigmoid(gi[:, ENC:2 * ENC] + gh[:, ENC:2 * ENC])
        n = jnp.tanh(gi[:, 2 * ENC:] + r * gh[:, 2 * ENC:])
        return (1.0 - z) * n + z * h

    h_f = jnp.zeros((AB, ENC), f32)
    h_b = jnp.zeros((AB, ENC), f32)
    for t in range(IN_LENGTH):
        h_f = gru_step(h_f, t, WihTf_ref, WhhTf_ref, bihf_ref, bhhf_ref)
        h_b = gru_step(h_b, IN_LENGTH - 1 - t, WihTb_ref, WhhTb_ref,
                       bihb_ref, bhhb_ref)
    seq = _lrelu(jnp.dot(((h_f + h_b) * 0.5).astype(bf),
                         WmlpT_ref[...].astype(bf),
                         preferred_element_type=f32) + bmlp_ref[...])

    bsp1 = bsp1_ref[...]
    P0 = jnp.dot(Fg_ref[...].astype(bf), Wsp1T_ref[...].astype(bf),
                 preferred_element_type=f32)
    r0 = _lrelu(jnp.dot(_lrelu(bsp1), Wsp2T_ref[...],
                        preferred_element_type=f32) + bsp2_ref[...])
    parts = []
    for g in range(SB // SG):
        Pg = P0[g * SG * G:(g + 1) * SG * G, :].reshape(SG, G, ENC)
        D = Pg[:, None, :, :] - Pg[:, :, None, :]
        A = _lrelu(D.reshape(SG * G * G, ENC) + bsp1)
        A2 = _lrelu(jnp.dot(A.astype(bf), Wsp2T_ref[...].astype(bf),
                            preferred_element_type=f32) + bsp2_ref[...])
        S = A2.reshape(SG * G, G, ENC // 2).sum(axis=1)
        parts.append(S)
    pooled = (jnp.concatenate(parts, axis=0) - r0) * (1.0 / (G - 1))

    full = jnp.concatenate([seq, pooled], axis=1)
    full_ref[...] = full
    x1 = _lrelu(jnp.dot(full.astype(bf), Wop1T_ref[...].astype(bf),
                        preferred_element_type=f32)
                + bop1_ref[...])
    logits_ref[...] = (jnp.dot(x1.astype(bf), Wop2T_ref[...].astype(bf),
                               preferred_element_type=f32)
                       + bop2_ref[...])


@functools.partial(jax.jit, static_argnums=())
def kernel(scene, condition, hero_pos_index, nbrs_pos_index, index_div,
           W_emb, b_emb, Wih_f, Whh_f, bih_f, bhh_f,
           Wih_b, Whh_b, bih_b, bhh_b,
           W_mlp, b_mlp, W_sp1, b_sp1, W_sp2, b_sp2,
           W_op1, b_op1, W_op2, b_op2):
    f32 = jnp.float32
    T = IN_LENGTH
    F = scene.reshape(N, 2 * T)
    F_pad = jnp.pad(F, ((0, 0), (0, 128 - 2 * T)))
    Fg = _sc_gather(F_pad, index_div.reshape(-1))

    eye = jnp.eye(T, dtype=f32)
    pad64 = lambda v: jnp.pad(v, (0, 128 - EMB))
    Wbig = jnp.concatenate(
        [jnp.kron(eye, pad64(W_emb[:, 0])[None, :]),
         jnp.kron(eye, pad64(W_emb[:, 1])[None, :])], axis=0)
    bemb = jnp.tile(pad64(b_emb), (1, T)).reshape(1, T * 128)
    padih = lambda w: jnp.pad(w.T, ((0, 128 - EMB), (0, 0)))

    Wop2T = jnp.zeros((EMB, 128), f32).at[:, :CLASS_NUM].set(W_op2.T)
    bop2 = jnp.zeros((1, 128), f32).at[0, :CLASS_NUM].set(b_op2)

    row2 = lambda v: v.reshape(1, -1)
    grid = (B // SB,)
    blk = lambda shape: pl.BlockSpec(shape, lambda i: (0, 0))
    out = pl.pallas_call(
        _fused_kernel,
        grid=grid,
        in_specs=[
            pl.BlockSpec((AB, 2 * T), lambda i: (i, 0)),
            pl.BlockSpec((AB, 128), lambda i: (i, 0)),
            blk((2 * T, T * 128)),
            blk((1, T * 128)),
            blk((128, 3 * ENC)), blk((ENC, 3 * ENC)),
            blk((1, 3 * ENC)), blk((1, 3 * ENC)),
            blk((128, 3 * ENC)), blk((ENC, 3 * ENC)),
            blk((1, 3 * ENC)), blk((1, 3 * ENC)),
            blk((ENC, ENC // 2)), blk((1, ENC // 2)),
            blk((128, ENC)), blk((1, ENC)),
            blk((ENC, ENC // 2)), blk((1, ENC // 2)),
            blk((ENC, EMB)), blk((1, EMB)),
            blk((EMB, 128)), blk((1, 128)),
        ],
        out_specs=[
            pl.BlockSpec((AB, 128), lambda i: (i, 0)),
            pl.BlockSpec((AB, ENC), lambda i: (i, 0)),
        ],
        out_shape=[
            jax.ShapeDtypeStruct((N, 128), f32),
            jax.ShapeDtypeStruct((N, ENC), f32),
        ],
    )(F, Fg, Wbig, bemb,
      padih(Wih_f), Whh_f.T, row2(bih_f), row2(bhh_f),
      padih(Wih_b), Whh_b.T, row2(bih_b), row2(bhh_b),
      W_mlp.T, row2(b_mlp),
      jnp.pad(W_sp1.T, ((0, 128 - 2 * T), (0, 0))), row2(b_sp1),
      W_sp2.T, row2(b_sp2), W_op1.T, row2(b_op1),
      Wop2T, bop2)
    logits_pad, full_enc = out
    return (logits_pad[:, :CLASS_NUM], full_enc)


# speedup vs baseline: 1.0120x; 1.0120x over previous
import functools

import jax
import jax.numpy as jnp
from jax import lax
from jax.experimental import pallas as pl
from jax.experimental.pallas import tpu as pltpu
from jax.experimental.pallas import tpu_sc as plsc

IN_LENGTH = 16
ENC = 256
EMB = 64
CLASS_NUM = 3
B = 256
G = 32
N = B * G

SB = 64
AB = SB * G
SG = 16


_SC_NC = 2
_SC_NS = 16
_SC_NW = _SC_NC * _SC_NS


def _sc_gather(table, idx):
    rows, d = table.shape
    b_per_w = rows // _SC_NW
    mesh = plsc.VectorSubcoreMesh(core_axis_name="c", subcore_axis_name="s")

    @functools.partial(
        pl.kernel, mesh=mesh,
        out_type=jax.ShapeDtypeStruct((rows, d), table.dtype),
        scratch_types=[
            pltpu.VMEM((b_per_w,), jnp.int32),
            pltpu.VMEM((b_per_w, d), table.dtype),
            pltpu.SemaphoreType.DMA,
        ],
    )
    def k(table_hbm, idx_hbm, out_hbm, idx_v, rows_v, sem):
        wid = lax.axis_index("s") * _SC_NC + lax.axis_index("c")
        base = wid * b_per_w
        pltpu.sync_copy(idx_hbm.at[pl.ds(base, b_per_w)], idx_v)
        pltpu.async_copy(table_hbm.at[idx_v], rows_v, sem).wait()
        pltpu.sync_copy(rows_v, out_hbm.at[pl.ds(base, b_per_w)])

    return k(table, idx)


def _lrelu(v):
    return jnp.maximum(v, 0.1 * v)


def _sigmoid(v):
    return 0.5 * jnp.tanh(0.5 * v) + 0.5


def _fused_kernel(F_ref, Fg_ref, Wbig_ref, bemb_ref,
                  WihTf_ref, WhhTf_ref, bihf_ref, bhhf_ref,
                  WihTb_ref, WhhTb_ref, bihb_ref, bhhb_ref,
                  WmlpT_ref, bmlp_ref, Wsp1T_ref, bsp1_ref,
                  Wsp2T_ref, bsp2_ref, Wop1T_ref, bop1_ref,
                  Wop2T_ref, bop2_ref,
                  logits_ref, full_ref):
    f32 = jnp.float32
    bf = jnp.bfloat16
    F = F_ref[...]
    emb = _lrelu(jnp.dot(F.astype(bf), Wbig_ref[...].astype(bf),
                         preferred_element_type=f32)
                 + bemb_ref[...])
    emb = emb.astype(bf)

    def gru_step(h, t, WihT_ref, WhhT_ref, bih_ref, bhh_ref):
        e_t = emb[:, t * EMB:(t + 1) * EMB]
        gi = jnp.dot(e_t, WihT_ref[...].astype(bf),
                     preferred_element_type=f32) + bih_ref[...]
        gh = jnp.dot(h.astype(bf), WhhT_ref[...].astype(bf),
                     preferred_element_type=f32) + bhh_ref[...]
        r = _sigmoid(gi[:, :ENC] + gh[:, :ENC])
        z = _sigmoid(gi[:, ENC:2 * ENC] + gh[:, ENC:2 * ENC])
        n = jnp.tanh(gi[:, 2 * ENC:] + r * gh[:, 2 * ENC:])
        return (1.0 - z) * n + z * h

    h_f = jnp.zeros((AB, ENC), f32)
    h_b = jnp.zeros((AB, ENC), f32)
    for t in range(IN_LENGTH):
        h_f = gru_step(h_f, t, WihTf_ref, WhhTf_ref, bihf_ref, bhhf_ref)
        h_b = gru_step(h_b, IN_LENGTH - 1 - t, WihTb_ref, WhhTb_ref,
                       bihb_ref, bhhb_ref)
    seq = _lrelu(jnp.dot(((h_f + h_b) * 0.5).astype(bf),
                         WmlpT_ref[...].astype(bf),
                         preferred_element_type=f32) + bmlp_ref[...])

    bsp1 = bsp1_ref[...]
    P0 = jnp.dot(Fg_ref[...].astype(bf), Wsp1T_ref[...].astype(bf),
                 preferred_element_type=f32)
    r0 = _lrelu(jnp.dot(_lrelu(bsp1), Wsp2T_ref[...],
                        preferred_element_type=f32) + bsp2_ref[...])
    parts = []
    for g in range(SB // SG):
        Pg = P0[g * SG * G:(g + 1) * SG * G, :].reshape(SG, G, ENC)
        D = Pg[:, None, :, :] - Pg[:, :, None, :]
        A = _lrelu(D.reshape(SG * G * G, ENC) + bsp1)
        A2 = _lrelu(jnp.dot(A.astype(bf), Wsp2T_ref[...].astype(bf),
                            preferred_element_type=f32) + bsp2_ref[...])
        S = A2.reshape(SG * G, G, ENC // 2).sum(axis=1)
        parts.append(S)
    pooled = (jnp.concatenate(parts, axis=0) - r0) * (1.0 / (G - 1))

    full = jnp.concatenate([seq, pooled], axis=1)
    full_ref[...] = full
    x1 = _lrelu(jnp.dot(full.astype(bf), Wop1T_ref[...].astype(bf),
                        preferred_element_type=f32)
                + bop1_ref[...])
    logits_ref[...] = (jnp.dot(x1.astype(bf), Wop2T_ref[...].astype(bf),
                               preferred_element_type=f32)
                       + bop2_ref[...])


@functools.partial(jax.jit, static_argnums=())
def kernel(scene, condition, hero_pos_index, nbrs_pos_index, index_div,
           W_emb, b_emb, Wih_f, Whh_f, bih_f, bhh_f,
           Wih_b, Whh_b, bih_b, bhh_b,
           W_mlp, b_mlp, W_sp1, b_sp1, W_sp2, b_sp2,
           W_op1, b_op1, W_op2, b_op2):
    f32 = jnp.float32
    T = IN_LENGTH
    F = scene.reshape(N, 2 * T)
    F_pad = jnp.pad(F, ((0, 0), (0, 128 - 2 * T)))
    Fg = _sc_gather(F_pad, index_div.reshape(-1))

    eye = jnp.eye(T, dtype=f32)
    Wbig = jnp.concatenate(
        [jnp.kron(eye, W_emb[:, 0][None, :]),
         jnp.kron(eye, W_emb[:, 1][None, :])], axis=0)
    bemb = jnp.tile(b_emb, (1, T)).reshape(1, T * EMB)

    Wop2T = jnp.zeros((EMB, 128), f32).at[:, :CLASS_NUM].set(W_op2.T)
    bop2 = jnp.zeros((1, 128), f32).at[0, :CLASS_NUM].set(b_op2)

    row2 = lambda v: v.reshape(1, -1)
    grid = (B // SB,)
    blk = lambda shape: pl.BlockSpec(shape, lambda i: (0, 0))
    out = pl.pallas_call(
        _fused_kernel,
        grid=grid,
        in_specs=[
            pl.BlockSpec((AB, 2 * T), lambda i: (i, 0)),
            pl.BlockSpec((AB, 128), lambda i: (i, 0)),
            blk((2 * T, T * EMB)),
            blk((1, T * EMB)),
            blk((EMB, 3 * ENC)), blk((ENC, 3 * ENC)),
            blk((1, 3 * ENC)), blk((1, 3 * ENC)),
            blk((EMB, 3 * ENC)), blk((ENC, 3 * ENC)),
            blk((1, 3 * ENC)), blk((1, 3 * ENC)),
            blk((ENC, ENC // 2)), blk((1, ENC // 2)),
            blk((128, ENC)), blk((1, ENC)),
            blk((ENC, ENC // 2)), blk((1, ENC // 2)),
            blk((ENC, EMB)), blk((1, EMB)),
            blk((EMB, 128)), blk((1, 128)),
        ],
        out_specs=[
            pl.BlockSpec((AB, 128), lambda i: (i, 0)),
            pl.BlockSpec((AB, ENC), lambda i: (i, 0)),
        ],
        out_shape=[
            jax.ShapeDtypeStruct((N, 128), f32),
            jax.ShapeDtypeStruct((N, ENC), f32),
        ],
    )(F, Fg, Wbig, bemb,
      Wih_f.T, Whh_f.T, row2(bih_f), row2(bhh_f),
      Wih_b.T, Whh_b.T, row2(bih_b), row2(bhh_b),
      W_mlp.T, row2(b_mlp),
      jnp.pad(W_sp1.T, ((0, 128 - 2 * T), (0, 0))), row2(b_sp1),
      W_sp2.T, row2(b_sp2), W_op1.T, row2(b_op1),
      Wop2T, bop2)
    logits_pad, full_enc = out
    return (logits_pad[:, :CLASS_NUM], full_enc)


# speedup vs baseline: 1.1019x; 1.0888x over previous
import functools

import jax
import jax.numpy as jnp
from jax import lax
from jax.experimental import pallas as pl
from jax.experimental.pallas import tpu as pltpu
from jax.experimental.pallas import tpu_sc as plsc

IN_LENGTH = 16
ENC = 256
EMB = 64
CLASS_NUM = 3
B = 256
G = 32
N = B * G

SB = 64
AB = SB * G
SG = 8


_SC_NC = 2
_SC_NS = 16
_SC_NW = _SC_NC * _SC_NS


def _sc_gather(table, idx):
    rows, d = table.shape
    b_per_w = rows // _SC_NW
    mesh = plsc.VectorSubcoreMesh(core_axis_name="c", subcore_axis_name="s")

    @functools.partial(
        pl.kernel, mesh=mesh,
        out_type=jax.ShapeDtypeStruct((rows, d), table.dtype),
        scratch_types=[
            pltpu.VMEM((b_per_w,), jnp.int32),
            pltpu.VMEM((b_per_w, d), table.dtype),
            pltpu.SemaphoreType.DMA,
        ],
    )
    def k(table_hbm, idx_hbm, out_hbm, idx_v, rows_v, sem):
        wid = lax.axis_index("s") * _SC_NC + lax.axis_index("c")
        base = wid * b_per_w
        pltpu.sync_copy(idx_hbm.at[pl.ds(base, b_per_w)], idx_v)
        pltpu.async_copy(table_hbm.at[idx_v], rows_v, sem).wait()
        pltpu.sync_copy(rows_v, out_hbm.at[pl.ds(base, b_per_w)])

    return k(table, idx)


def _lrelu(v):
    return jnp.maximum(v, 0.1 * v)


def _fused_kernel(F_ref, Fg_ref, Wbig_ref, bemb_ref,
                  WihTf_ref, WhhTf_ref, brzf_ref, binf_ref, bhnf_ref,
                  WihTb_ref, WhhTb_ref, brzb_ref, binb_ref, bhnb_ref,
                  WmlpT_ref, bmlp_ref, Wsp1T_ref, bsp1_ref,
                  Wsp2T_ref, bsp2_ref, Wop1T_ref, bop1_ref,
                  Wop2T_ref, bop2_ref,
                  logits_ref, full_ref):
    f32 = jnp.float32
    bf = jnp.bfloat16
    F = F_ref[...]
    emb = _lrelu(jnp.dot(F.astype(bf), Wbig_ref[...].astype(bf),
                         preferred_element_type=f32)
                 + bemb_ref[...])
    emb = emb.astype(bf)

    def gru_step(h, t, WihT_ref, WhhT_ref, brz_ref, bin_ref, bhn_ref):
        e_t = emb[:, t * EMB:(t + 1) * EMB]
        gi = jnp.dot(e_t, WihT_ref[...].astype(bf),
                     preferred_element_type=f32)
        gh = jnp.dot(h.astype(bf), WhhT_ref[...].astype(bf),
                     preferred_element_type=f32)
        trz = jnp.tanh(gi[:, :2 * ENC] + gh[:, :2 * ENC] + brz_ref[...])
        r = 0.5 * trz[:, :ENC] + 0.5
        tz = trz[:, ENC:]
        n = jnp.tanh(gi[:, 2 * ENC:] + bin_ref[...]
                     + r * (gh[:, 2 * ENC:] + bhn_ref[...]))
        return 0.5 * ((n + h) + tz * (h - n))

    h_f = jnp.zeros((AB, ENC), f32)
    h_b = jnp.zeros((AB, ENC), f32)
    for t in range(IN_LENGTH):
        h_f = gru_step(h_f, t, WihTf_ref, WhhTf_ref, brzf_ref,
                       binf_ref, bhnf_ref)
        h_b = gru_step(h_b, IN_LENGTH - 1 - t, WihTb_ref, WhhTb_ref,
                       brzb_ref, binb_ref, bhnb_ref)
    seq = _lrelu(jnp.dot(((h_f + h_b) * 0.5).astype(bf),
                         WmlpT_ref[...].astype(bf),
                         preferred_element_type=f32) + bmlp_ref[...])

    bsp1 = bsp1_ref[...]
    P0 = jnp.dot(Fg_ref[...].astype(bf), Wsp1T_ref[...].astype(bf),
                 preferred_element_type=f32)
    r0 = _lrelu(jnp.dot(_lrelu(bsp1), Wsp2T_ref[...],
                        preferred_element_type=f32) + bsp2_ref[...])
    Pj = (P0 + bsp1).astype(bf)
    Pi = P0.astype(bf)
    parts = []
    for g in range(SB // SG):
        sl = slice(g * SG * G, (g + 1) * SG * G)
        Pjg = Pj[sl].reshape(SG, G, ENC)
        Pig = Pi[sl].reshape(SG, G, ENC)
        D = Pjg[:, None, :, :] - Pig[:, :, None, :]
        A = _lrelu(D.reshape(SG * G * G, ENC))
        A2 = _lrelu(jnp.dot(A, Wsp2T_ref[...].astype(bf),
                            preferred_element_type=f32) + bsp2_ref[...])
        S = A2.reshape(SG * G, G, ENC // 2).sum(axis=1)
        parts.append(S)
    pooled = (jnp.concatenate(parts, axis=0) - r0) * (1.0 / (G - 1))

    full = jnp.concatenate([seq, pooled], axis=1)
    full_ref[...] = full
    x1 = _lrelu(jnp.dot(full.astype(bf), Wop1T_ref[...].astype(bf),
                        preferred_element_type=f32)
                + bop1_ref[...])
    logits_ref[...] = (jnp.dot(x1.astype(bf), Wop2T_ref[...].astype(bf),
                               preferred_element_type=f32)
                       + bop2_ref[...])


@functools.partial(jax.jit, static_argnums=())
def kernel(scene, condition, hero_pos_index, nbrs_pos_index, index_div,
           W_emb, b_emb, Wih_f, Whh_f, bih_f, bhh_f,
           Wih_b, Whh_b, bih_b, bhh_b,
           W_mlp, b_mlp, W_sp1, b_sp1, W_sp2, b_sp2,
           W_op1, b_op1, W_op2, b_op2):
    f32 = jnp.float32
    T = IN_LENGTH
    F = scene.reshape(N, 2 * T)
    F_pad = jnp.pad(F, ((0, 0), (0, 128 - 2 * T)))
    Fg = _sc_gather(F_pad, index_div.reshape(-1))

    eye = jnp.eye(T, dtype=f32)
    Wbig = jnp.concatenate(
        [jnp.kron(eye, W_emb[:, 0][None, :]),
         jnp.kron(eye, W_emb[:, 1][None, :])], axis=0)
    bemb = jnp.tile(b_emb, (1, T)).reshape(1, T * EMB)

    Wop2T = jnp.zeros((EMB, 128), f32).at[:, :CLASS_NUM].set(W_op2.T)
    bop2 = jnp.zeros((1, 128), f32).at[0, :CLASS_NUM].set(b_op2)

    gsc = jnp.concatenate([jnp.full((2 * ENC,), 0.5, f32),
                           jnp.ones((ENC,), f32)])[None, :]
    def gru_params(Wih, Whh, bih, bhh):
        return (Wih.T * gsc, Whh.T * gsc,
                (0.5 * (bih[:2 * ENC] + bhh[:2 * ENC])).reshape(1, -1),
                bih[2 * ENC:].reshape(1, -1), bhh[2 * ENC:].reshape(1, -1))

    row2 = lambda v: v.reshape(1, -1)
    grid = (B // SB,)
    blk = lambda shape: pl.BlockSpec(shape, lambda i: (0, 0))
    out = pl.pallas_call(
        _fused_kernel,
        grid=grid,
        in_specs=[
            pl.BlockSpec((AB, 2 * T), lambda i: (i, 0)),
            pl.BlockSpec((AB, 128), lambda i: (i, 0)),
            blk((2 * T, T * EMB)),
            blk((1, T * EMB)),
            blk((EMB, 3 * ENC)), blk((ENC, 3 * ENC)),
            blk((1, 2 * ENC)), blk((1, ENC)), blk((1, ENC)),
            blk((EMB, 3 * ENC)), blk((ENC, 3 * ENC)),
            blk((1, 2 * ENC)), blk((1, ENC)), blk((1, ENC)),
            blk((ENC, ENC // 2)), blk((1, ENC // 2)),
            blk((128, ENC)), blk((1, ENC)),
            blk((ENC, ENC // 2)), blk((1, ENC // 2)),
            blk((ENC, EMB)), blk((1, EMB)),
            blk((EMB, 128)), blk((1, 128)),
        ],
        out_specs=[
            pl.BlockSpec((AB, 128), lambda i: (i, 0)),
            pl.BlockSpec((AB, ENC), lambda i: (i, 0)),
        ],
        out_shape=[
            jax.ShapeDtypeStruct((N, 128), f32),
            jax.ShapeDtypeStruct((N, ENC), f32),
        ],
    )(F, Fg, Wbig, bemb,
      *gru_params(Wih_f, Whh_f, bih_f, bhh_f),
      *gru_params(Wih_b, Whh_b, bih_b, bhh_b),
      W_mlp.T, row2(b_mlp),
      jnp.pad(W_sp1.T, ((0, 128 - 2 * T), (0, 0))), row2(b_sp1),
      W_sp2.T, row2(b_sp2), W_op1.T, row2(b_op1),
      Wop2T, bop2)
    logits_pad, full_enc = out
    return (logits_pad[:, :CLASS_NUM], full_enc)
